# baseline (speedup 1.0000x reference)
import jax
import jax.numpy as jnp
from jax.experimental import pallas as pl
from jax.experimental.pallas import tpu as pltpu
from jax.experimental.pallas import tpu_sc as plsc

_TC_BLK = 512
_STC = 3072
_BLKR = 8
_VEC = 16


def _tc_body(x_ref, e_ref, o_ref):
    o_ref[...] = x_ref[...] + e_ref[...][None, :, :]


def kernel(inputs, embeddings):
    batch, seq_len, dim = inputs.shape
    ssc = seq_len - _STC
    nseq_r = seq_len // _BLKR
    stc_b = _STC // _BLKR
    flat = inputs.reshape(batch * seq_len, dim)

    emb = pltpu.with_memory_space_constraint(
        embeddings[:seq_len], pltpu.MemorySpace.HBM
    )
    tc_full = pl.pallas_call(
        _tc_body,
        grid=(_STC // _TC_BLK,),
        in_specs=[
            pl.BlockSpec((batch, _TC_BLK, dim), lambda i: (0, i, 0)),
            pl.BlockSpec((_TC_BLK, dim), lambda i: (i, 0)),
        ],
        out_specs=pl.BlockSpec((batch, _TC_BLK, dim), lambda i: (0, i, 0)),
        out_shape=jax.ShapeDtypeStruct((batch, seq_len, dim), inputs.dtype),
    )(inputs, emb)

    mesh = plsc.VectorSubcoreMesh(core_axis_name="c", subcore_axis_name="s")

    @pl.kernel(
        out_type=jax.ShapeDtypeStruct((batch * ssc, dim), inputs.dtype),
        mesh=mesh,
    )
    def sc_add(x_hbm, e_hbm, o_hbm):
        def body(x_vmem, e_vmem, o_vmem):
            @pl.loop(0, _BLKR)
            def _(r):
                @plsc.parallel_loop(0, dim, step=_VEC, unroll=8)
                def _(c):
                    slc = (pl.ds(r, 1), pl.ds(c, _VEC))
                    o_vmem.at[*slc][...] = x_vmem.at[*slc][...] + e_vmem.at[*slc][...]

        pltpu.emit_pipeline(
            body,
            grid=(batch, ssc // _BLKR),
            in_specs=[
                pl.BlockSpec(
                    (_BLKR, dim), index_map=lambda b, i: (b * nseq_r + stc_b + i, 0)
                ),
                pl.BlockSpec((_BLKR, dim), index_map=lambda b, i: (stc_b + i, 0)),
            ],
            out_specs=[
                pl.BlockSpec(
                    (_BLKR, dim), index_map=lambda b, i: (b * (ssc // _BLKR) + i, 0)
                )
            ],
            core_axis_name=("c", "s"),
            dimension_semantics=(pltpu.PARALLEL, pltpu.PARALLEL),
        )(x_hbm, e_hbm, o_hbm)

    sc_piece = sc_add(flat, embeddings).reshape(batch, ssc, dim)
    return jax.lax.dynamic_update_slice(tc_full, sc_piece, (0, _STC, 0))


# speedup vs baseline: 1.9314x; 1.9314x over previous
import jax
import jax.numpy as jnp
from jax.experimental import pallas as pl
from jax.experimental.pallas import tpu as pltpu


def _add_body(x_ref, e_ref, o_ref):
    o_ref[...] = x_ref[...] + e_ref[...][None, :, :]


def kernel(inputs, embeddings):
    batch, seq_len, dim = inputs.shape
    blk = 1024
    grid = (seq_len // blk,)
    emb = pltpu.with_memory_space_constraint(
        embeddings[:seq_len], pltpu.MemorySpace.HBM
    )
    return pl.pallas_call(
        _add_body,
        grid=grid,
        in_specs=[
            pl.BlockSpec((batch, blk, dim), lambda i: (0, i, 0)),
            pl.BlockSpec((blk, dim), lambda i: (i, 0)),
        ],
        out_specs=pl.BlockSpec((batch, blk, dim), lambda i: (0, i, 0)),
        out_shape=jax.ShapeDtypeStruct((batch, seq_len, dim), inputs.dtype),
    )(inputs, emb)


# speedup vs baseline: 1.9568x; 1.0132x over previous
import jax
import jax.numpy as jnp
from jax.experimental import pallas as pl
from jax.experimental.pallas import tpu as pltpu


def _add_body(x_ref, e_ref, o_ref):
    o_ref[...] = x_ref[...] + e_ref[...][None, :, :]


def kernel(inputs, embeddings):
    batch, seq_len, dim = inputs.shape
    blk = 512
    grid = (seq_len // blk,)
    emb = pltpu.with_memory_space_constraint(
        embeddings[:seq_len], pltpu.MemorySpace.HBM
    )
    return pl.pallas_call(
        _add_body,
        grid=grid,
        in_specs=[
            pl.BlockSpec((batch, blk, dim), lambda i: (0, i, 0)),
            pl.BlockSpec((blk, dim), lambda i: (i, 0)),
        ],
        out_specs=pl.BlockSpec((batch, blk, dim), lambda i: (0, i, 0)),
        out_shape=jax.ShapeDtypeStruct((batch, seq_len, dim), inputs.dtype),
    )(inputs, emb)
